# Initial kernel scaffold; baseline (speedup 1.0000x reference)
import jax
import jax.numpy as jnp
from jax.experimental import pallas as pl


def kernel(x_data, ijk, W_mid, W_out):
    raise NotImplementedError("write your pallas kernel here")


# speedup vs baseline: 29.5083x; 29.5083x over previous
import functools

import jax
import jax.numpy as jnp
from jax import lax
from jax.experimental import pallas as pl
from jax.experimental.pallas import tpu as pltpu
from jax.experimental.pallas import tpu_sc as plsc

N = 32768
R = 64
S = 8
IN_CH = 256
MID_CH = 256
C = MID_CH // S
OUT_CH = 128
ROWS = N * S

NC, NS = 2, 16
NW = NC * NS
ROWS_PER_W = ROWS // NW
BATCH = 1024
NBATCH = ROWS_PER_W // BATCH
DMA_ROWS = 128
NDMA = BATCH // DMA_ROWS


def _mm1_body(x_ref, w_ref, o_ref):
    o_ref[...] = jnp.dot(x_ref[...], w_ref[...],
                         preferred_element_type=jnp.float32)


def _tc_matmul1(x, w):
    BM = 1024
    return pl.pallas_call(
        _mm1_body,
        grid=(N // BM,),
        in_specs=[
            pl.BlockSpec((BM, IN_CH), lambda m: (m, 0)),
            pl.BlockSpec((IN_CH, MID_CH), lambda m: (0, 0)),
        ],
        out_specs=pl.BlockSpec((BM, MID_CH), lambda m: (m, 0)),
        out_shape=jax.ShapeDtypeStruct((N, MID_CH), jnp.float32),
    )(x, w)


def _mm2_body(x_ref, w_ref, o_ref):
    o_ref[...] = jnp.dot(x_ref[...], w_ref[...],
                         preferred_element_type=jnp.float32)


def _tc_matmul2(x, w):
    BM = 4096
    return pl.pallas_call(
        _mm2_body,
        grid=(ROWS // BM,),
        in_specs=[
            pl.BlockSpec((BM, C), lambda m: (m, 0)),
            pl.BlockSpec((C, OUT_CH), lambda m: (0, 0)),
        ],
        out_specs=pl.BlockSpec((BM, OUT_CH), lambda m: (m, 0)),
        out_shape=jax.ShapeDtypeStruct((ROWS, OUT_CH), jnp.float32),
    )(x, w)


def _sc_scatter_body(h_ref, dst_ref, out_ref, rows_v, idx_v, sem):
    w = lax.axis_index("s") * NC + lax.axis_index("c")

    def batch_body(b, carry):
        base = (w * NBATCH + b) * BATCH
        pltpu.sync_copy(h_ref.at[pl.ds(base, BATCH)], rows_v)
        rowb = pl.multiple_of((w * NBATCH + b) * NDMA, 8)
        pltpu.sync_copy(dst_ref.at[pl.ds(rowb, NDMA)], idx_v)
        descs = []
        for j in range(NDMA):
            descs.append(pltpu.async_copy(
                rows_v.at[pl.ds(j * DMA_ROWS, DMA_ROWS)],
                out_ref.at[idx_v.at[j]],
                sem))
        for d in descs:
            d.wait()
        return carry

    lax.fori_loop(0, NBATCH, batch_body, 0)


def _sc_scatter(h2, dst2d):
    mesh = plsc.VectorSubcoreMesh(core_axis_name="c", subcore_axis_name="s",
                                  num_cores=NC, num_subcores=NS)
    f = pl.kernel(
        _sc_scatter_body,
        out_type=jax.ShapeDtypeStruct((ROWS, C), jnp.float32),
        mesh=mesh,
        scratch_types=[
            pltpu.VMEM((BATCH, C), jnp.float32),
            pltpu.VMEM((NDMA, DMA_ROWS), jnp.int32),
            pltpu.SemaphoreType.DMA,
        ],
        compiler_params=pltpu.CompilerParams(use_tc_tiling_on_sc=False),
    )
    return f(h2, dst2d)


def _dst_indices(ijk):
    ii = ijk[:, 0].astype(jnp.int32)
    jj = ijk[:, 1].astype(jnp.int32)
    ijkey = ii * R + jj
    cnt = jnp.zeros((R * R,), jnp.int32).at[ijkey].add(1)
    P = jnp.concatenate([jnp.zeros((1,), jnp.int32),
                         jnp.cumsum(cnt, dtype=jnp.int32)])
    Ai = jnp.take(P, ii * R)
    Ci = jnp.take(P, ii * R + R) - Ai
    Aij = jnp.take(P, ijkey)
    Cij = jnp.take(P, ijkey + 1) - Aij
    p = jnp.arange(N, dtype=jnp.int32)
    base0 = 8 * Ai + 4 * (Aij - Ai) + 2 * (p - Aij)
    l = jnp.arange(S, dtype=jnp.int32)
    di = (l >> 2) & 1
    dj = (l >> 1) & 1
    dk = l & 1
    dst = (base0[:, None] + di[None, :] * (4 * Ci)[:, None]
           + dj[None, :] * (2 * Cij)[:, None] + dk[None, :])
    return dst.reshape(ROWS // 128, 128)


def kernel(x_data, ijk, W_mid, W_out):
    dst2d = _dst_indices(ijk)
    h = _tc_matmul1(x_data, W_mid)
    hg = _sc_scatter(h.reshape(ROWS, C), dst2d)
    return _tc_matmul2(hg, W_out)


# speedup vs baseline: 120.5828x; 4.0864x over previous
import jax
import jax.numpy as jnp
from jax import lax
from jax.experimental import pallas as pl
from jax.experimental.pallas import tpu as pltpu
from jax.experimental.pallas import tpu_sc as plsc

N = 32768
R = 64
S = 8
IN_CH = 256
MID_CH = 256
C = MID_CH // S
OUT_CH = 128
ROWS = N * S

NC, NS = 2, 16
NW = NC * NS
PW = N // NW
NPT = N // NS
BATCH = 1024
NBATCH = (ROWS // NW) // BATCH
DMA_ROWS = 128
NDMA = BATCH // DMA_ROWS
NBINS = R * R
BPT = NBINS // NS
PLEN = NBINS + 16


def _mm_body(x_ref, w_ref, o_ref):
    o_ref[...] = jnp.dot(x_ref[...], w_ref[...],
                         preferred_element_type=jnp.float32)


def _tc_matmul1(x, w):
    BM = 1024
    return pl.pallas_call(
        _mm_body,
        grid=(N // BM,),
        in_specs=[
            pl.BlockSpec((BM, IN_CH), lambda m: (m, 0)),
            pl.BlockSpec((IN_CH, MID_CH), lambda m: (0, 0)),
        ],
        out_specs=pl.BlockSpec((BM, MID_CH), lambda m: (m, 0)),
        out_shape=jax.ShapeDtypeStruct((N, MID_CH), jnp.float32),
    )(x, w)


def _tc_matmul2(x, w):
    BM = 4096
    return pl.pallas_call(
        _mm_body,
        grid=(ROWS // BM,),
        in_specs=[
            pl.BlockSpec((BM, C), lambda m: (m, 0)),
            pl.BlockSpec((C, OUT_CH), lambda m: (0, 0)),
        ],
        out_specs=pl.BlockSpec((BM, OUT_CH), lambda m: (m, 0)),
        out_shape=jax.ShapeDtypeStruct((ROWS, OUT_CH), jnp.float32),
    )(x, w)


def _sc_body(ijk_ref, h_ref, out_ref,
             hist_v, ijk_v, comb_v, pfx_v, tot16_v, totb_v, pout_v, p_v,
             rows_v, idx_v, hists_sh, tot_sh, p_sh, sem):
    c = lax.axis_index("c")
    s = lax.axis_index("s")
    w = s * NC + c
    lanes = lax.iota(jnp.int32, 16)
    zeros16 = jnp.zeros((16,), jnp.int32)
    ones16 = jnp.ones((16,), jnp.int32)

    def zero_body(t, carry):
        hist_v[pl.ds(t * 16, 16)] = zeros16
        return carry
    lax.fori_loop(0, NBINS // 16, zero_body, 0)

    pltpu.sync_copy(ijk_ref.at[pl.ds(s * (NPT * 3), NPT * 3)],
                    ijk_v.at[pl.ds(0, NPT * 3)])

    def hist_body(t, carry):
        base3 = (t * 16 + lanes) * 3
        iv = plsc.load_gather(ijk_v, [base3])
        jv = plsc.load_gather(ijk_v, [base3 + 1])
        plsc.addupdate_scatter(hist_v, [iv * R + jv], ones16)
        return carry
    lax.fori_loop(0, NPT // 16, hist_body, 0)

    pltpu.sync_copy(hist_v, hists_sh.at[s])
    plsc.subcore_barrier()

    for r in range(NS):
        pltpu.sync_copy(hists_sh.at[r, pl.ds(s * BPT, BPT)], comb_v.at[r])

    def pref_body(t, tot):
        v = comb_v[0, pl.ds(t * 16, 16)]
        for r in range(1, NS):
            v = v + comb_v[r, pl.ds(t * 16, 16)]
        incl = plsc.cumsum(v)
        pfx_v[pl.ds(t * 16, 16)] = incl - v + tot
        return tot + jnp.sum(v)
    tile_total = lax.fori_loop(0, BPT // 16, pref_body, 0)

    totb_v[...] = jnp.full((16,), tile_total, jnp.int32)
    pltpu.sync_copy(totb_v, tot_sh.at[s])
    plsc.subcore_barrier()

    pltpu.sync_copy(tot_sh, tot16_v)
    tvec = plsc.load_gather(tot16_v, [lanes, zeros16])
    offs = jnp.sum(jnp.where(lanes < s, tvec, 0))

    def poff_body(t, carry):
        pout_v[pl.ds(t * 16, 16)] = pfx_v[pl.ds(t * 16, 16)] + offs
        return carry
    lax.fori_loop(0, BPT // 16, poff_body, 0)
    pltpu.sync_copy(pout_v, p_sh.at[pl.ds(s * BPT, BPT)])

    @pl.when(s == NS - 1)
    def _():
        totb_v[...] = jnp.full((16,), N, jnp.int32)
        pltpu.sync_copy(totb_v, p_sh.at[pl.ds(NBINS, 16)])
    plsc.subcore_barrier()

    pltpu.sync_copy(p_sh, p_v)

    pltpu.sync_copy(ijk_ref.at[pl.ds(w * (PW * 3), PW * 3)],
                    ijk_v.at[pl.ds(0, PW * 3)])

    def batch_body(b, carry):
        rowbase = (w * NBATCH + b) * BATCH
        pltpu.sync_copy(h_ref.at[pl.ds(rowbase, BATCH)], rows_v)
        for g in range(NDMA):
            pg = b * (BATCH // S) + g * 16 + lanes
            base3 = pg * 3
            iv = plsc.load_gather(ijk_v, [base3])
            jv = plsc.load_gather(ijk_v, [base3 + 1])
            key = iv * R + jv
            i64 = iv * R
            Ai = plsc.load_gather(p_v, [i64])
            Bi = plsc.load_gather(p_v, [i64 + R])
            Aij = plsc.load_gather(p_v, [key])
            Bij = plsc.load_gather(p_v, [key + 1])
            p = w * PW + pg
            base0 = 8 * Ai + 4 * (Aij - Ai) + 2 * (p - Aij)
            ci4 = 4 * (Bi - Ai)
            cij2 = 2 * (Bij - Aij)
            cols = lanes * 8
            for l in range(S):
                di, dj, dk = (l >> 2) & 1, (l >> 1) & 1, l & 1
                dst = base0 + di * ci4 + dj * cij2 + dk
                plsc.store_scatter(idx_v, [jnp.full((16,), g, jnp.int32),
                                           cols + l], dst)
        descs = []
        for j in range(NDMA):
            descs.append(pltpu.async_copy(
                rows_v.at[pl.ds(j * DMA_ROWS, DMA_ROWS)],
                out_ref.at[idx_v.at[j]],
                sem))
        for d in descs:
            d.wait()
        return carry

    lax.fori_loop(0, NBATCH, batch_body, 0)


def _sc_scatter(ijk_flat, h2):
    mesh = plsc.VectorSubcoreMesh(core_axis_name="c", subcore_axis_name="s",
                                  num_cores=NC, num_subcores=NS)
    f = pl.kernel(
        _sc_body,
        out_type=jax.ShapeDtypeStruct((ROWS, C), jnp.float32),
        mesh=mesh,
        scratch_types=[
            pltpu.VMEM((NBINS,), jnp.int32),
            pltpu.VMEM((NPT * 3,), jnp.int32),
            pltpu.VMEM((NS, BPT), jnp.int32),
            pltpu.VMEM((BPT,), jnp.int32),
            pltpu.VMEM((NS, 16), jnp.int32),
            pltpu.VMEM((16,), jnp.int32),
            pltpu.VMEM((BPT,), jnp.int32),
            pltpu.VMEM((PLEN,), jnp.int32),
            pltpu.VMEM((BATCH, C), jnp.float32),
            pltpu.VMEM((NDMA, DMA_ROWS), jnp.int32),
            pltpu.VMEM_SHARED((NS, NBINS), jnp.int32),
            pltpu.VMEM_SHARED((NS, 16), jnp.int32),
            pltpu.VMEM_SHARED((PLEN,), jnp.int32),
            pltpu.SemaphoreType.DMA,
        ],
        compiler_params=pltpu.CompilerParams(use_tc_tiling_on_sc=False,
                                             needs_layout_passes=False),
    )
    return f(ijk_flat, h2)


def kernel(x_data, ijk, W_mid, W_out):
    ijk_flat = ijk.reshape(-1).astype(jnp.int32)
    h = _tc_matmul1(x_data, W_mid)
    hg = _sc_scatter(ijk_flat, h.reshape(ROWS, C))
    return _tc_matmul2(hg, W_out)


# speedup vs baseline: 157.6244x; 1.3072x over previous
import jax
import jax.numpy as jnp
from jax import lax
from jax.experimental import pallas as pl
from jax.experimental.pallas import tpu as pltpu
from jax.experimental.pallas import tpu_sc as plsc

N = 32768
R = 64
S = 8
IN_CH = 256
MID_CH = 256
C = MID_CH // S
OUT_CH = 128
ROWS = N * S

NC, NS = 2, 16
NW = NC * NS
PW = N // NW
NPT = N // NS
BATCH = 512
NBATCH = (ROWS // NW) // BATCH
DMA_ROWS = 128
NDMA = BATCH // DMA_ROWS
NBINS = R * R
BPT = NBINS // NS
PLEN = NBINS + 16


def _tc_body(x_ref, wm_ref, wo_ref, o_ref):
    hb = jnp.dot(x_ref[...], wm_ref[...], preferred_element_type=jnp.float32)
    for l in range(S):
        o_ref[:, l, :] = jnp.dot(hb[:, l * C:(l + 1) * C], wo_ref[...],
                                 preferred_element_type=jnp.float32)


def _tc_dense(x, wm, wo):
    BM = 1024
    return pl.pallas_call(
        _tc_body,
        grid=(N // BM,),
        in_specs=[
            pl.BlockSpec((BM, IN_CH), lambda m: (m, 0)),
            pl.BlockSpec((IN_CH, MID_CH), lambda m: (0, 0)),
            pl.BlockSpec((C, OUT_CH), lambda m: (0, 0)),
        ],
        out_specs=pl.BlockSpec((BM, S, OUT_CH), lambda m: (m, 0, 0)),
        out_shape=jax.ShapeDtypeStruct((N, S, OUT_CH), jnp.float32),
    )(x, wm, wo)


def _sc_body(ijk_ref, od_ref, out_ref,
             hist_v, ijk_v, comb_v, pfx_v, tot16_v, totb_v, pout_v, p_v,
             rows_v, idx_v, hists_sh, tot_sh, p_sh, sem):
    c = lax.axis_index("c")
    s = lax.axis_index("s")
    w = s * NC + c
    lanes = lax.iota(jnp.int32, 16)
    zeros16 = jnp.zeros((16,), jnp.int32)
    ones16 = jnp.ones((16,), jnp.int32)

    def zero_body(t, carry):
        hist_v[pl.ds(t * 16, 16)] = zeros16
        return carry
    lax.fori_loop(0, NBINS // 16, zero_body, 0)

    pltpu.sync_copy(ijk_ref.at[pl.ds(s * (NPT * 3), NPT * 3)],
                    ijk_v.at[pl.ds(0, NPT * 3)])

    def hist_body(t, carry):
        base3 = (t * 16 + lanes) * 3
        iv = plsc.load_gather(ijk_v, [base3])
        jv = plsc.load_gather(ijk_v, [base3 + 1])
        plsc.addupdate_scatter(hist_v, [iv * R + jv], ones16)
        return carry
    lax.fori_loop(0, NPT // 16, hist_body, 0)

    pltpu.sync_copy(hist_v, hists_sh.at[s])
    plsc.subcore_barrier()

    for r in range(NS):
        pltpu.sync_copy(hists_sh.at[r, pl.ds(s * BPT, BPT)], comb_v.at[r])

    def pref_body(t, tot):
        v = comb_v[0, pl.ds(t * 16, 16)]
        for r in range(1, NS):
            v = v + comb_v[r, pl.ds(t * 16, 16)]
        incl = plsc.cumsum(v)
        pfx_v[pl.ds(t * 16, 16)] = incl - v + tot
        return tot + jnp.sum(v)
    tile_total = lax.fori_loop(0, BPT // 16, pref_body, 0)

    totb_v[...] = jnp.full((16,), tile_total, jnp.int32)
    pltpu.sync_copy(totb_v, tot_sh.at[s])
    plsc.subcore_barrier()

    pltpu.sync_copy(tot_sh, tot16_v)
    tvec = plsc.load_gather(tot16_v, [lanes, zeros16])
    offs = jnp.sum(jnp.where(lanes < s, tvec, 0))

    def poff_body(t, carry):
        pout_v[pl.ds(t * 16, 16)] = pfx_v[pl.ds(t * 16, 16)] + offs
        return carry
    lax.fori_loop(0, BPT // 16, poff_body, 0)
    pltpu.sync_copy(pout_v, p_sh.at[pl.ds(s * BPT, BPT)])

    @pl.when(s == NS - 1)
    def _():
        totb_v[...] = jnp.full((16,), N, jnp.int32)
        pltpu.sync_copy(totb_v, p_sh.at[pl.ds(NBINS, 16)])
    plsc.subcore_barrier()

    pltpu.sync_copy(p_sh, p_v)

    pltpu.sync_copy(ijk_ref.at[pl.ds(w * (PW * 3), PW * 3)],
                    ijk_v.at[pl.ds(0, PW * 3)])

    def batch_body(b, carry):
        rowbase = (w * NBATCH + b) * BATCH
        pltpu.sync_copy(od_ref.at[pl.ds(rowbase, BATCH)], rows_v)
        for g in range(NDMA):
            pg = b * (BATCH // S) + g * 16 + lanes
            base3 = pg * 3
            iv = plsc.load_gather(ijk_v, [base3])
            jv = plsc.load_gather(ijk_v, [base3 + 1])
            key = iv * R + jv
            i64 = iv * R
            Ai = plsc.load_gather(p_v, [i64])
            Bi = plsc.load_gather(p_v, [i64 + R])
            Aij = plsc.load_gather(p_v, [key])
            Bij = plsc.load_gather(p_v, [key + 1])
            p = w * PW + pg
            base0 = 8 * Ai + 4 * (Aij - Ai) + 2 * (p - Aij)
            ci4 = 4 * (Bi - Ai)
            cij2 = 2 * (Bij - Aij)
            cols = lanes * 8
            for l in range(S):
                di, dj, dk = (l >> 2) & 1, (l >> 1) & 1, l & 1
                dst = base0 + di * ci4 + dj * cij2 + dk
                plsc.store_scatter(idx_v, [jnp.full((16,), g, jnp.int32),
                                           cols + l], dst)
        descs = []
        for j in range(NDMA):
            descs.append(pltpu.async_copy(
                rows_v.at[pl.ds(j * DMA_ROWS, DMA_ROWS)],
                out_ref.at[idx_v.at[j]],
                sem))
        for d in descs:
            d.wait()
        return carry

    lax.fori_loop(0, NBATCH, batch_body, 0)


def _sc_scatter(ijk_flat, od):
    mesh = plsc.VectorSubcoreMesh(core_axis_name="c", subcore_axis_name="s",
                                  num_cores=NC, num_subcores=NS)
    f = pl.kernel(
        _sc_body,
        out_type=jax.ShapeDtypeStruct((ROWS, OUT_CH), jnp.float32),
        mesh=mesh,
        scratch_types=[
            pltpu.VMEM((NBINS,), jnp.int32),
            pltpu.VMEM((NPT * 3,), jnp.int32),
            pltpu.VMEM((NS, BPT), jnp.int32),
            pltpu.VMEM((BPT,), jnp.int32),
            pltpu.VMEM((NS, 16), jnp.int32),
            pltpu.VMEM((16,), jnp.int32),
            pltpu.VMEM((BPT,), jnp.int32),
            pltpu.VMEM((PLEN,), jnp.int32),
            pltpu.VMEM((BATCH, OUT_CH), jnp.float32),
            pltpu.VMEM((NDMA, DMA_ROWS), jnp.int32),
            pltpu.VMEM_SHARED((NS, NBINS), jnp.int32),
            pltpu.VMEM_SHARED((NS, 16), jnp.int32),
            pltpu.VMEM_SHARED((PLEN,), jnp.int32),
            pltpu.SemaphoreType.DMA,
        ],
        compiler_params=pltpu.CompilerParams(needs_layout_passes=False),
    )
    return f(ijk_flat, od)


def kernel(x_data, ijk, W_mid, W_out):
    ijk_flat = ijk.reshape(-1).astype(jnp.int32)
    od = _tc_dense(x_data, W_mid, W_out)
    return _sc_scatter(ijk_flat, od.reshape(ROWS, OUT_CH))
